# baseline (speedup 1.0000x reference)
import jax
import jax.numpy as jnp
from jax import lax
from jax.experimental import pallas as pl
from jax.experimental.pallas import tpu as pltpu
from jax.experimental.pallas import tpu_sc as plsc

B = 16384
D = 64
N_ROWS = 1000000
T0 = 999936
NTAIL = N_ROWS - T0
NC = 2
NS = 16
L = 16
NW = NC * NS
BPW = B // NW
CHUNK = 128
NCHUNK = BPW // CHUNK
NBUF = 4


JBLK = 131072
_CHUNKS = [(jb * JBLK, JBLK) for jb in range(7)] + [(7 * JBLK, T0 - 7 * JBLK)]
_STEPS = [(t, a, off, ln)
          for t in range(2) for a in range(D // 8) for (off, ln) in _CHUNKS]


_CVT = 512


def _detile_body(wu_ref, wi_ref, ou_ref, oi_ref, buf_ref, bufb_ref,
                 sem_in, sem_out):
    def cin(r):
        t, a, off, ln = _STEPS[r]
        src = (wu_ref if t == 0 else wi_ref).at[pl.ds(8 * a, 8), pl.ds(off, ln)]
        dst = buf_ref.at[r % NBUF].at[pl.ds(0, 8), pl.ds(0, ln)]
        return pltpu.make_async_copy(src, dst, sem_in)

    def convert(r):
        t, a, off, ln = _STEPS[r]
        fb = buf_ref.at[r % NBUF]
        bb = bufb_ref.at[pl.ds((r % NBUF) * 8 * JBLK, 8 * JBLK)]

        def cvt(i, carry):
            sl = pl.ds(i * _CVT, _CVT)
            for s in range(8):
                bb[pl.ds(s * JBLK + i * _CVT, _CVT)] = (
                    fb[s, sl].astype(jnp.bfloat16))
            return carry

        lax.fori_loop(0, (ln + _CVT - 1) // _CVT, cvt, 0)

    def couts(r):
        t, a, off, ln = _STEPS[r]
        ref = ou_ref if t == 0 else oi_ref
        return [
            pltpu.make_async_copy(
                bufb_ref.at[pl.ds(((r % NBUF) * 8 + s) * JBLK, ln)],
                ref.at[pl.ds((8 * a + s) * T0 + off, ln)], sem_out)
            for s in range(8)
        ]

    total = len(_STEPS)
    for r in range(NBUF - 1):
        cin(r).start()
    for r in range(total):
        cin(r).wait()
        convert(r)
        for cp in couts(r):
            cp.start()
        nxt = r + NBUF - 1
        if nxt < total:
            if r >= 1:
                for cp in couts(r - 1):
                    cp.wait()
            cin(nxt).start()
    for r in range(max(0, total - NBUF), total):
        for cp in couts(r):
            cp.wait()


def _detile(wuT2d, wiT2d):
    return pl.pallas_call(
        _detile_body,
        in_specs=[
            pl.BlockSpec(memory_space=pl.ANY),
            pl.BlockSpec(memory_space=pl.ANY),
        ],
        out_specs=[
            pl.BlockSpec(memory_space=pl.ANY),
            pl.BlockSpec(memory_space=pl.ANY),
        ],
        out_shape=[
            jax.ShapeDtypeStruct((D * T0,), jnp.bfloat16),
            jax.ShapeDtypeStruct((D * T0,), jnp.bfloat16),
        ],
        scratch_shapes=[
            pltpu.VMEM((NBUF, 8, JBLK), jnp.float32),
            pltpu.VMEM((NBUF * 8 * JBLK,), jnp.bfloat16),
            pltpu.SemaphoreType.DMA,
            pltpu.SemaphoreType.DMA,
        ],
    )(wuT2d, wiT2d)


def _sc_body(uidx_hbm, iidx_hbm, wuT_hbm, wiT_hbm, mu_hbm, bu_hbm, bi_hbm,
             tu_hbm, ti_hbm, out_hbm, uidx_v, iidx_v, cu_v, ci_v, u_all,
             v_all, bu_v, bi_v, mu_v, tu_v, ti_v, out_v, sem):
    wid = lax.axis_index("s") * NC + lax.axis_index("c")
    base = wid * BPW

    pltpu.sync_copy(uidx_hbm.at[pl.ds(base, BPW)], uidx_v)
    pltpu.sync_copy(iidx_hbm.at[pl.ds(base, BPW)], iidx_v)
    pltpu.sync_copy(mu_hbm, mu_v.at[pl.ds(0, 1)])
    pltpu.sync_copy(tu_hbm, tu_v)
    pltpu.sync_copy(ti_hbm, ti_v)

    def clamp_body(t, carry):
        sl = pl.ds(t * L, L)
        cu_v[sl] = jnp.minimum(uidx_v[sl], T0 - 1) >> 1
        ci_v[sl] = jnp.minimum(iidx_v[sl], T0 - 1) >> 1
        return carry

    lax.fori_loop(0, BPW // L, clamp_body, 0)

    bias_copies = []
    for k in range(NCHUNK):
        dst = pl.ds(k * CHUNK, CHUNK)
        idx_u = uidx_v.at[pl.ds(k * CHUNK, CHUNK)]
        idx_i = iidx_v.at[pl.ds(k * CHUNK, CHUNK)]
        bias_copies.append(pltpu.async_copy(bu_hbm.at[idx_u], bu_v.at[dst], sem))
        bias_copies.append(pltpu.async_copy(bi_hbm.at[idx_i], bi_v.at[dst], sem))
    for cp in bias_copies:
        cp.wait()

    mu_s = mu_v[...][0]
    lane = lax.iota(jnp.int32, L)

    T0_2 = T0 // 2
    HI = jnp.int32(-65536)
    NG = CHUNK // L

    def _decode(c, g, k, pu, pi):
        sl = pl.ds(g * L, L)
        ui = u_all[c, sl]
        vi = v_all[c, sl]
        uc = plsc.bitcast(jnp.where(pu[g], ui & HI, ui << 16), jnp.float32)
        vc = plsc.bitcast(jnp.where(pi[g], vi & HI, vi << 16), jnp.float32)
        return uc, vc

    def chunk_body(k, carry):
        idx_u = cu_v.at[pl.ds(k * CHUNK, CHUNK)]
        idx_i = ci_v.at[pl.ds(k * CHUNK, CHUNK)]
        copies = []
        for c in range(D):
            src_u = wuT_hbm.at[pl.ds(c * T0_2, T0_2)]
            src_i = wiT_hbm.at[pl.ds(c * T0_2, T0_2)]
            copies.append(pltpu.async_copy(src_u.at[idx_u], u_all.at[c], sem))
            copies.append(pltpu.async_copy(src_i.at[idx_i], v_all.at[c], sem))
        for cp in copies:
            cp.wait()

        pu = []
        pi = []
        any_tail = jnp.zeros((), jnp.bool_)
        for g in range(NG):
            sl = pl.ds(k * CHUNK + g * L, L)
            uj = uidx_v[sl]
            ij = iidx_v[sl]
            pu.append((uj & 1) == 1)
            pi.append((ij & 1) == 1)
            any_tail = jnp.logical_or(
                any_tail, jnp.any((uj >= T0) | (ij >= T0)))
        pu = tuple(pu)
        pi = tuple(pi)

        def c_body(c, carry2):
            accs, pu, pi = carry2
            new = []
            for g in range(NG):
                uc, vc = _decode(c, g, k, pu, pi)
                new.append(accs[g] + uc * vc)
            return (tuple(new), pu, pi)

        accs, _, _ = lax.fori_loop(
            0, D, c_body,
            (tuple(jnp.zeros((L,), jnp.float32) for _ in range(NG)), pu, pi))
        for g in range(NG):
            sl = pl.ds(k * CHUNK + g * L, L)
            out_v[sl] = accs[g] + bu_v[sl] + bi_v[sl] + mu_s

        @pl.when(any_tail)
        def _fix():
            def fat_body(c, carry2):
                accs2, pu, pi = carry2
                new = []
                for g in range(NG):
                    uc, vc = _decode(c, g, k, pu, pi)
                    sl = pl.ds(k * CHUNK + g * L, L)
                    ue = uidx_v[sl] - T0
                    ie = iidx_v[sl] - T0
                    ut = plsc.load_gather(
                        tu_v, [c * NTAIL + jnp.clip(ue, 0, NTAIL - 1)])
                    vt = plsc.load_gather(
                        ti_v, [c * NTAIL + jnp.clip(ie, 0, NTAIL - 1)])
                    uc = jnp.where(ue >= 0, ut, uc)
                    vc = jnp.where(ie >= 0, vt, vc)
                    new.append(accs2[g] + uc * vc)
                return (tuple(new), pu, pi)

            accs2, _, _ = lax.fori_loop(
                0, D, fat_body,
                (tuple(jnp.zeros((L,), jnp.float32) for _ in range(NG)),
                 pu, pi))
            for g in range(NG):
                sl = pl.ds(k * CHUNK + g * L, L)
                out_v[sl] = accs2[g] + bu_v[sl] + bi_v[sl] + mu_s

        return carry

    lax.fori_loop(0, NCHUNK, chunk_body, 0)
    pltpu.sync_copy(out_v, out_hbm.at[pl.ds(base, BPW)])


def kernel(x, W_user, W_item, mu, b_user, b_item):
    uidx = x[:, 0]
    iidx = x[:, 1]
    wuT_bf, wiT_bf = _detile(W_user.T, W_item.T)
    wuT = lax.bitcast_convert_type(wuT_bf.reshape(-1, 2), jnp.int32)
    wiT = lax.bitcast_convert_type(wiT_bf.reshape(-1, 2), jnp.int32)
    tails_u = W_user[T0:, :].T.reshape(-1)
    tails_i = W_item[T0:, :].T.reshape(-1)
    mu_flat = mu.reshape(1)
    bu = b_user.reshape(-1)
    bi = b_item.reshape(-1)

    mesh = plsc.VectorSubcoreMesh(core_axis_name="c", subcore_axis_name="s",
                                  num_cores=NC, num_subcores=NS)
    k = pl.kernel(
        _sc_body,
        out_type=jax.ShapeDtypeStruct((B,), jnp.float32),
        mesh=mesh,
        scratch_types=[
            pltpu.VMEM((BPW,), jnp.int32),
            pltpu.VMEM((BPW,), jnp.int32),
            pltpu.VMEM((BPW,), jnp.int32),
            pltpu.VMEM((BPW,), jnp.int32),
            pltpu.VMEM((D, CHUNK), jnp.int32),
            pltpu.VMEM((D, CHUNK), jnp.int32),
            pltpu.VMEM((BPW,), jnp.float32),
            pltpu.VMEM((BPW,), jnp.float32),
            pltpu.VMEM((L,), jnp.float32),
            pltpu.VMEM((D * NTAIL,), jnp.float32),
            pltpu.VMEM((D * NTAIL,), jnp.float32),
            pltpu.VMEM((BPW,), jnp.float32),
            pltpu.SemaphoreType.DMA,
        ],
        compiler_params=pltpu.CompilerParams(needs_layout_passes=False,
                                             use_tc_tiling_on_sc=False),
    )
    return k(uidx, iidx, wuT, wiT, mu_flat, bu, bi, tails_u, tails_i)


# speedup vs baseline: 85.1116x; 85.1116x over previous
import jax
import jax.numpy as jnp
from jax import lax
from jax.experimental import pallas as pl
from jax.experimental.pallas import tpu as pltpu
from jax.experimental.pallas import tpu_sc as plsc

B = 16384
D = 64
N_ROWS = 1000000
T0 = 999936
NTAIL = N_ROWS - T0
NC = 2
NS = 16
L = 16
NW = NC * NS
BPW = B // NW
CHUNK = 128
NCHUNK = BPW // CHUNK
NBUF = 4


JBLK = 131072
_CHUNKS = [(jb * JBLK, JBLK) for jb in range(7)] + [(7 * JBLK, T0 - 7 * JBLK)]
_STEPS = [(t, a, off, ln)
          for t in range(2) for a in range(D // 8) for (off, ln) in _CHUNKS]


def _detile_body(wu_ref, wi_ref, ou_ref, oi_ref, buf_ref, sem_in, sem_out):
    def cin(r):
        t, a, off, ln = _STEPS[r]
        src = (wu_ref if t == 0 else wi_ref).at[pl.ds(8 * a, 8), pl.ds(off, ln)]
        dst = buf_ref.at[r % NBUF].at[pl.ds(0, 8), pl.ds(0, ln)]
        return pltpu.make_async_copy(src, dst, sem_in)

    def couts(r):
        t, a, off, ln = _STEPS[r]
        ref = ou_ref if t == 0 else oi_ref
        return [
            pltpu.make_async_copy(
                buf_ref.at[r % NBUF].at[s].at[pl.ds(0, ln)],
                ref.at[pl.ds((8 * a + s) * T0 + off, ln)], sem_out)
            for s in range(8)
        ]

    total = len(_STEPS)
    for r in range(NBUF - 1):
        cin(r).start()
    for r in range(total):
        cin(r).wait()
        for cp in couts(r):
            cp.start()
        nxt = r + NBUF - 1
        if nxt < total:
            if r >= 1:
                for cp in couts(r - 1):
                    cp.wait()
            cin(nxt).start()
    for r in range(max(0, total - NBUF), total):
        for cp in couts(r):
            cp.wait()


def _detile(wuT2d, wiT2d):
    return pl.pallas_call(
        _detile_body,
        in_specs=[
            pl.BlockSpec(memory_space=pl.ANY),
            pl.BlockSpec(memory_space=pl.ANY),
        ],
        out_specs=[
            pl.BlockSpec(memory_space=pl.ANY),
            pl.BlockSpec(memory_space=pl.ANY),
        ],
        out_shape=[
            jax.ShapeDtypeStruct((D * T0,), jnp.float32),
            jax.ShapeDtypeStruct((D * T0,), jnp.float32),
        ],
        scratch_shapes=[
            pltpu.VMEM((NBUF, 8, JBLK), jnp.float32),
            pltpu.SemaphoreType.DMA,
            pltpu.SemaphoreType.DMA,
        ],
    )(wuT2d, wiT2d)


def _sc_body(uidx_hbm, iidx_hbm, wuT_hbm, wiT_hbm, mu_hbm, bu_hbm, bi_hbm,
             tu_hbm, ti_hbm, out_hbm, uidx_v, iidx_v, cu_v, ci_v, u_all,
             v_all, bu_v, bi_v, mu_v, tu_v, ti_v, out_v, sem):
    wid = lax.axis_index("s") * NC + lax.axis_index("c")
    base = wid * BPW

    pltpu.sync_copy(uidx_hbm.at[pl.ds(base, BPW)], uidx_v)
    pltpu.sync_copy(iidx_hbm.at[pl.ds(base, BPW)], iidx_v)
    pltpu.sync_copy(mu_hbm, mu_v.at[pl.ds(0, 1)])
    pltpu.sync_copy(tu_hbm, tu_v)
    pltpu.sync_copy(ti_hbm, ti_v)

    def clamp_body(t, carry):
        sl = pl.ds(t * L, L)
        cu_v[sl] = jnp.minimum(uidx_v[sl], T0 - 1)
        ci_v[sl] = jnp.minimum(iidx_v[sl], T0 - 1)
        return carry

    lax.fori_loop(0, BPW // L, clamp_body, 0)

    bias_copies = []
    for k in range(NCHUNK):
        dst = pl.ds(k * CHUNK, CHUNK)
        idx_u = uidx_v.at[pl.ds(k * CHUNK, CHUNK)]
        idx_i = iidx_v.at[pl.ds(k * CHUNK, CHUNK)]
        bias_copies.append(pltpu.async_copy(bu_hbm.at[idx_u], bu_v.at[dst], sem))
        bias_copies.append(pltpu.async_copy(bi_hbm.at[idx_i], bi_v.at[dst], sem))
    for cp in bias_copies:
        cp.wait()

    mu_s = mu_v[...][0]
    lane = lax.iota(jnp.int32, L)

    def chunk_body(k, carry):
        idx_u = cu_v.at[pl.ds(k * CHUNK, CHUNK)]
        idx_i = ci_v.at[pl.ds(k * CHUNK, CHUNK)]
        copies = []
        for c in range(D):
            src_u = wuT_hbm.at[pl.ds(c * T0, T0)]
            src_i = wiT_hbm.at[pl.ds(c * T0, T0)]
            copies.append(pltpu.async_copy(src_u.at[idx_u], u_all.at[c], sem))
            copies.append(pltpu.async_copy(src_i.at[idx_i], v_all.at[c], sem))
        for cp in copies:
            cp.wait()

        def c_body(c, accs):
            return tuple(
                accs[j] + u_all[c, pl.ds(j * L, L)] * v_all[c, pl.ds(j * L, L)]
                for j in range(CHUNK // L)
            )

        accs = lax.fori_loop(
            0, D, c_body,
            tuple(jnp.zeros((L,), jnp.float32) for _ in range(CHUNK // L)))
        any_tail = jnp.zeros((), jnp.bool_)
        for j in range(CHUNK // L):
            sl = pl.ds(k * CHUNK + j * L, L)
            out_v[sl] = accs[j] + bu_v[sl] + bi_v[sl] + mu_s
            any_tail = jnp.logical_or(
                any_tail,
                jnp.any((uidx_v[sl] >= T0) | (iidx_v[sl] >= T0)))

        @pl.when(any_tail)
        def _fix():
            def fat_body(c, accs2):
                cc = jnp.full((L,), c, jnp.int32)
                new = []
                for j in range(CHUNK // L):
                    sl = pl.ds(k * CHUNK + j * L, L)
                    uj = uidx_v[sl]
                    ij = iidx_v[sl]
                    ue = uj - T0
                    ie = ij - T0
                    uc = u_all[c, pl.ds(j * L, L)]
                    vc = v_all[c, pl.ds(j * L, L)]
                    ut = plsc.load_gather(
                        tu_v, [c * NTAIL + jnp.clip(ue, 0, NTAIL - 1)])
                    vt = plsc.load_gather(
                        ti_v, [c * NTAIL + jnp.clip(ie, 0, NTAIL - 1)])
                    uc = jnp.where(ue >= 0, ut, uc)
                    vc = jnp.where(ie >= 0, vt, vc)
                    new.append(accs2[j] + uc * vc)
                return tuple(new)

            accs2 = lax.fori_loop(
                0, D, fat_body,
                tuple(jnp.zeros((L,), jnp.float32) for _ in range(CHUNK // L)))
            for j in range(CHUNK // L):
                sl = pl.ds(k * CHUNK + j * L, L)
                out_v[sl] = accs2[j] + bu_v[sl] + bi_v[sl] + mu_s

        return carry

    lax.fori_loop(0, NCHUNK, chunk_body, 0)
    pltpu.sync_copy(out_v, out_hbm.at[pl.ds(base, BPW)])


def kernel(x, W_user, W_item, mu, b_user, b_item):
    uidx = x[:, 0]
    iidx = x[:, 1]
    wuT, wiT = _detile(W_user.T, W_item.T)
    tails_u = W_user[T0:, :].T.reshape(-1)
    tails_i = W_item[T0:, :].T.reshape(-1)
    mu_flat = mu.reshape(1)
    bu = b_user.reshape(-1)
    bi = b_item.reshape(-1)

    mesh = plsc.VectorSubcoreMesh(core_axis_name="c", subcore_axis_name="s",
                                  num_cores=NC, num_subcores=NS)
    k = pl.kernel(
        _sc_body,
        out_type=jax.ShapeDtypeStruct((B,), jnp.float32),
        mesh=mesh,
        scratch_types=[
            pltpu.VMEM((BPW,), jnp.int32),
            pltpu.VMEM((BPW,), jnp.int32),
            pltpu.VMEM((BPW,), jnp.int32),
            pltpu.VMEM((BPW,), jnp.int32),
            pltpu.VMEM((D, CHUNK), jnp.float32),
            pltpu.VMEM((D, CHUNK), jnp.float32),
            pltpu.VMEM((BPW,), jnp.float32),
            pltpu.VMEM((BPW,), jnp.float32),
            pltpu.VMEM((L,), jnp.float32),
            pltpu.VMEM((D * NTAIL,), jnp.float32),
            pltpu.VMEM((D * NTAIL,), jnp.float32),
            pltpu.VMEM((BPW,), jnp.float32),
            pltpu.SemaphoreType.DMA,
        ],
        compiler_params=pltpu.CompilerParams(needs_layout_passes=False,
                                             use_tc_tiling_on_sc=False),
    )
    return k(uidx, iidx, wuT, wiT, mu_flat, bu, bi, tails_u, tails_i)


# speedup vs baseline: 85.1560x; 1.0005x over previous
import jax
import jax.numpy as jnp
from jax import lax
from jax.experimental import pallas as pl
from jax.experimental.pallas import tpu as pltpu
from jax.experimental.pallas import tpu_sc as plsc

B = 16384
D = 64
N_ROWS = 1000000
T0 = 999936
NTAIL = N_ROWS - T0
NC = 2
NS = 16
L = 16
NW = NC * NS
BPW = B // NW
CHUNK = 128
NCHUNK = BPW // CHUNK
NBUF = 6


JBLK = 131072
_CHUNKS = [(jb * JBLK, JBLK) for jb in range(7)] + [(7 * JBLK, T0 - 7 * JBLK)]
_STEPS = [(t, a, off, ln)
          for t in range(2) for a in range(D // 8) for (off, ln) in _CHUNKS]


def _detile_body(wu_ref, wi_ref, ou_ref, oi_ref, buf_ref, sem_in, sem_out):
    def cin(r):
        t, a, off, ln = _STEPS[r]
        src = (wu_ref if t == 0 else wi_ref).at[pl.ds(8 * a, 8), pl.ds(off, ln)]
        dst = buf_ref.at[r % NBUF].at[pl.ds(0, 8), pl.ds(0, ln)]
        return pltpu.make_async_copy(src, dst, sem_in)

    def couts(r):
        t, a, off, ln = _STEPS[r]
        ref = ou_ref if t == 0 else oi_ref
        return [
            pltpu.make_async_copy(
                buf_ref.at[r % NBUF].at[s].at[pl.ds(0, ln)],
                ref.at[pl.ds((8 * a + s) * T0 + off, ln)], sem_out)
            for s in range(8)
        ]

    total = len(_STEPS)
    for r in range(NBUF - 1):
        cin(r).start()
    for r in range(total):
        cin(r).wait()
        for cp in couts(r):
            cp.start()
        nxt = r + NBUF - 1
        if nxt < total:
            if r >= 1:
                for cp in couts(r - 1):
                    cp.wait()
            cin(nxt).start()
    for r in range(max(0, total - NBUF), total):
        for cp in couts(r):
            cp.wait()


def _detile(wuT2d, wiT2d):
    return pl.pallas_call(
        _detile_body,
        in_specs=[
            pl.BlockSpec(memory_space=pl.ANY),
            pl.BlockSpec(memory_space=pl.ANY),
        ],
        out_specs=[
            pl.BlockSpec(memory_space=pl.ANY),
            pl.BlockSpec(memory_space=pl.ANY),
        ],
        out_shape=[
            jax.ShapeDtypeStruct((D * T0,), jnp.float32),
            jax.ShapeDtypeStruct((D * T0,), jnp.float32),
        ],
        scratch_shapes=[
            pltpu.VMEM((NBUF, 8, JBLK), jnp.float32),
            pltpu.SemaphoreType.DMA,
            pltpu.SemaphoreType.DMA,
        ],
    )(wuT2d, wiT2d)


def _sc_body(uidx_hbm, iidx_hbm, wuT_hbm, wiT_hbm, mu_hbm, bu_hbm, bi_hbm,
             tu_hbm, ti_hbm, out_hbm, uidx_v, iidx_v, cu_v, ci_v, u_all,
             v_all, bu_v, bi_v, mu_v, tu_v, ti_v, out_v, sem):
    wid = lax.axis_index("s") * NC + lax.axis_index("c")
    base = wid * BPW

    pltpu.sync_copy(uidx_hbm.at[pl.ds(base, BPW)], uidx_v)
    pltpu.sync_copy(iidx_hbm.at[pl.ds(base, BPW)], iidx_v)
    pltpu.sync_copy(mu_hbm, mu_v.at[pl.ds(0, 1)])
    pltpu.sync_copy(tu_hbm, tu_v)
    pltpu.sync_copy(ti_hbm, ti_v)

    def clamp_body(t, carry):
        sl = pl.ds(t * L, L)
        cu_v[sl] = jnp.minimum(uidx_v[sl], T0 - 1)
        ci_v[sl] = jnp.minimum(iidx_v[sl], T0 - 1)
        return carry

    lax.fori_loop(0, BPW // L, clamp_body, 0)

    bias_copies = []
    for k in range(NCHUNK):
        dst = pl.ds(k * CHUNK, CHUNK)
        idx_u = uidx_v.at[pl.ds(k * CHUNK, CHUNK)]
        idx_i = iidx_v.at[pl.ds(k * CHUNK, CHUNK)]
        bias_copies.append(pltpu.async_copy(bu_hbm.at[idx_u], bu_v.at[dst], sem))
        bias_copies.append(pltpu.async_copy(bi_hbm.at[idx_i], bi_v.at[dst], sem))
    for cp in bias_copies:
        cp.wait()

    mu_s = mu_v[...][0]
    lane = lax.iota(jnp.int32, L)

    def chunk_body(k, carry):
        idx_u = cu_v.at[pl.ds(k * CHUNK, CHUNK)]
        idx_i = ci_v.at[pl.ds(k * CHUNK, CHUNK)]
        copies = []
        for c in range(D):
            src_u = wuT_hbm.at[pl.ds(c * T0, T0)]
            src_i = wiT_hbm.at[pl.ds(c * T0, T0)]
            copies.append(pltpu.async_copy(src_u.at[idx_u], u_all.at[c], sem))
            copies.append(pltpu.async_copy(src_i.at[idx_i], v_all.at[c], sem))
        for cp in copies:
            cp.wait()

        def c_body(c, accs):
            return tuple(
                accs[j] + u_all[c, pl.ds(j * L, L)] * v_all[c, pl.ds(j * L, L)]
                for j in range(CHUNK // L)
            )

        accs = lax.fori_loop(
            0, D, c_body,
            tuple(jnp.zeros((L,), jnp.float32) for _ in range(CHUNK // L)))
        any_tail = jnp.zeros((), jnp.bool_)
        for j in range(CHUNK // L):
            sl = pl.ds(k * CHUNK + j * L, L)
            out_v[sl] = accs[j] + bu_v[sl] + bi_v[sl] + mu_s
            any_tail = jnp.logical_or(
                any_tail,
                jnp.any((uidx_v[sl] >= T0) | (iidx_v[sl] >= T0)))

        @pl.when(any_tail)
        def _fix():
            def fat_body(c, accs2):
                cc = jnp.full((L,), c, jnp.int32)
                new = []
                for j in range(CHUNK // L):
                    sl = pl.ds(k * CHUNK + j * L, L)
                    uj = uidx_v[sl]
                    ij = iidx_v[sl]
                    ue = uj - T0
                    ie = ij - T0
                    uc = u_all[c, pl.ds(j * L, L)]
                    vc = v_all[c, pl.ds(j * L, L)]
                    ut = plsc.load_gather(
                        tu_v, [c * NTAIL + jnp.clip(ue, 0, NTAIL - 1)])
                    vt = plsc.load_gather(
                        ti_v, [c * NTAIL + jnp.clip(ie, 0, NTAIL - 1)])
                    uc = jnp.where(ue >= 0, ut, uc)
                    vc = jnp.where(ie >= 0, vt, vc)
                    new.append(accs2[j] + uc * vc)
                return tuple(new)

            accs2 = lax.fori_loop(
                0, D, fat_body,
                tuple(jnp.zeros((L,), jnp.float32) for _ in range(CHUNK // L)))
            for j in range(CHUNK // L):
                sl = pl.ds(k * CHUNK + j * L, L)
                out_v[sl] = accs2[j] + bu_v[sl] + bi_v[sl] + mu_s

        return carry

    lax.fori_loop(0, NCHUNK, chunk_body, 0)
    pltpu.sync_copy(out_v, out_hbm.at[pl.ds(base, BPW)])


def kernel(x, W_user, W_item, mu, b_user, b_item):
    uidx = x[:, 0]
    iidx = x[:, 1]
    wuT, wiT = _detile(W_user.T, W_item.T)
    tails_u = W_user[T0:, :].T.reshape(-1)
    tails_i = W_item[T0:, :].T.reshape(-1)
    mu_flat = mu.reshape(1)
    bu = b_user[:, 0]
    bi = b_item[:, 0]

    mesh = plsc.VectorSubcoreMesh(core_axis_name="c", subcore_axis_name="s",
                                  num_cores=NC, num_subcores=NS)
    k = pl.kernel(
        _sc_body,
        out_type=jax.ShapeDtypeStruct((B,), jnp.float32),
        mesh=mesh,
        scratch_types=[
            pltpu.VMEM((BPW,), jnp.int32),
            pltpu.VMEM((BPW,), jnp.int32),
            pltpu.VMEM((BPW,), jnp.int32),
            pltpu.VMEM((BPW,), jnp.int32),
            pltpu.VMEM((D, CHUNK), jnp.float32),
            pltpu.VMEM((D, CHUNK), jnp.float32),
            pltpu.VMEM((BPW,), jnp.float32),
            pltpu.VMEM((BPW,), jnp.float32),
            pltpu.VMEM((L,), jnp.float32),
            pltpu.VMEM((D * NTAIL,), jnp.float32),
            pltpu.VMEM((D * NTAIL,), jnp.float32),
            pltpu.VMEM((BPW,), jnp.float32),
            pltpu.SemaphoreType.DMA,
        ],
        compiler_params=pltpu.CompilerParams(needs_layout_passes=False,
                                             use_tc_tiling_on_sc=False),
    )
    return k(uidx, iidx, wuT, wiT, mu_flat, bu, bi, tails_u, tails_i)
